# Initial kernel scaffold; baseline (speedup 1.0000x reference)
import jax
import jax.numpy as jnp
from jax.experimental import pallas as pl


def kernel(query, key, value, query_pos, ref_pts_cam, spatial_shapes, bev_mask, W_value, b_value, W_off, b_off, W_attn, b_attn, W_out, b_out):
    raise NotImplementedError("write your pallas kernel here")


# speedup vs baseline: 21.3281x; 21.3281x over previous
import functools

import jax
import jax.numpy as jnp
from jax import lax
from jax.experimental import pallas as pl
from jax.experimental.pallas import tpu as pltpu
from jax.experimental.pallas import tpu_sc as plsc

N_CAM = 6
NQ = 10000
C = 128
H = 50
W = 100
HW = H * W
Y = 4
NUM_HEADS = 4
NUM_POINTS = 8
HEAD_DIM = C // NUM_HEADS

R_ROWS = N_CAM * NQ * NUM_HEADS
K_SAMP = NUM_POINTS * 4
TBL_ROWS = N_CAM * NUM_HEADS * HW


def _a1_body(v_ref, wv_ref, bv_ref, out_ref):
    x = jnp.dot(v_ref[0], wv_ref[...], preferred_element_type=jnp.float32)
    x = x + bv_ref[...]
    for h in range(NUM_HEADS):
        out_ref[0, h] = x[:, h * HEAD_DIM:(h + 1) * HEAD_DIM]


def _value_table(value, W_value, b_value):
    hw_blk = 1000
    grid = (N_CAM, HW // hw_blk)
    out = pl.pallas_call(
        _a1_body,
        grid=grid,
        in_specs=[
            pl.BlockSpec((1, hw_blk, C), lambda c, i: (c, i, 0)),
            pl.BlockSpec((C, C), lambda c, i: (0, 0)),
            pl.BlockSpec((1, C), lambda c, i: (0, 0)),
        ],
        out_specs=pl.BlockSpec((1, NUM_HEADS, hw_blk, HEAD_DIM),
                               lambda c, i: (c, 0, i, 0)),
        out_shape=jax.ShapeDtypeStruct((N_CAM, NUM_HEADS, HW, HEAD_DIM),
                                       jnp.float32),
    )(value, W_value.T, b_value.reshape(1, C))
    return out.reshape(TBL_ROWS, HEAD_DIM)


def _a2_body(q_ref, qp_ref, wo_ref, bo_ref, wa_ref, ba_ref, off_ref, aw_ref):
    q2 = q_ref[...] + qp_ref[...]
    off_ref[...] = jnp.dot(q2, wo_ref[...],
                           preferred_element_type=jnp.float32) + bo_ref[...]
    a = jnp.dot(q2, wa_ref[...], preferred_element_type=jnp.float32) + ba_ref[...]
    for h in range(NUM_HEADS):
        s = a[:, h * NUM_POINTS:(h + 1) * NUM_POINTS]
        m = jnp.max(s, axis=1, keepdims=True)
        e = jnp.exp(s - m)
        aw_ref[:, h * NUM_POINTS:(h + 1) * NUM_POINTS] = (
            e / jnp.sum(e, axis=1, keepdims=True))


def _proj_off_aw(query2d, query_pos2d, W_off, b_off, W_attn, b_attn):
    q_blk = 2000
    grid = (NQ // q_blk,)
    noff = NUM_HEADS * NUM_POINTS * 2
    naw = NUM_HEADS * NUM_POINTS
    off, aw = pl.pallas_call(
        _a2_body,
        grid=grid,
        in_specs=[
            pl.BlockSpec((q_blk, C), lambda i: (i, 0)),
            pl.BlockSpec((q_blk, C), lambda i: (i, 0)),
            pl.BlockSpec((C, noff), lambda i: (0, 0)),
            pl.BlockSpec((1, noff), lambda i: (0, 0)),
            pl.BlockSpec((C, naw), lambda i: (0, 0)),
            pl.BlockSpec((1, naw), lambda i: (0, 0)),
        ],
        out_specs=[
            pl.BlockSpec((q_blk, noff), lambda i: (i, 0)),
            pl.BlockSpec((q_blk, naw), lambda i: (i, 0)),
        ],
        out_shape=[
            jax.ShapeDtypeStruct((NQ, noff), jnp.float32),
            jax.ShapeDtypeStruct((NQ, naw), jnp.float32),
        ],
    )(query2d, query_pos2d, W_off.T, b_off.reshape(1, noff),
      W_attn.T, b_attn.reshape(1, naw))
    return off, aw


def _sample_idx_wt(off, aw, ref_pts_cam):
    off_r = off.reshape(NQ, NUM_HEADS, NUM_POINTS, 2)
    off_r = off_r / jnp.array([float(W), float(H)], jnp.float32)
    rr = ref_pts_cam.reshape(N_CAM, NQ, Y, 2)
    rr_p = rr[:, :, jnp.arange(NUM_POINTS) % Y, :]
    loc = rr_p[:, :, None, :, :] + off_r[None]
    x = loc[..., 0] * W - 0.5
    y = loc[..., 1] * H - 0.5
    x0 = jnp.floor(x)
    y0 = jnp.floor(y)
    wx1 = x - x0
    wx0 = 1.0 - wx1
    wy1 = y - y0
    wy0 = 1.0 - wy1
    xi = jnp.stack([x0, x0 + 1.0, x0, x0 + 1.0], axis=-1)
    yi = jnp.stack([y0, y0, y0 + 1.0, y0 + 1.0], axis=-1)
    wc = jnp.stack([wx0 * wy0, wx1 * wy0, wx0 * wy1, wx1 * wy1], axis=-1)
    valid = ((xi >= 0.0) & (xi <= float(W - 1))
             & (yi >= 0.0) & (yi <= float(H - 1))).astype(jnp.float32)
    xc = jnp.clip(xi, 0.0, float(W - 1)).astype(jnp.int32)
    yc = jnp.clip(yi, 0.0, float(H - 1)).astype(jnp.int32)
    lidx = yc * W + xc
    cam_h = (jnp.arange(N_CAM, dtype=jnp.int32)[:, None, None, None, None]
             * NUM_HEADS
             + jnp.arange(NUM_HEADS, dtype=jnp.int32)[None, None, :, None, None])
    gidx = cam_h * HW + lidx
    wt = aw.reshape(NQ, NUM_HEADS, NUM_POINTS)[None, :, :, :, None] * wc * valid
    return gidx.reshape(-1), wt.reshape(-1)


_SC_CH = 64
_SC_S = _SC_CH * K_SAMP
_SC_NCHUNKS = R_ROWS // _SC_CH


def _sc_gather_kernel(table_hbm, idx_hbm, wt_hbm, out_hbm,
                      idx_v, wt_v, rows_v, out_v, sem):
    info = plsc.get_sparse_core_info()
    nw = info.num_cores * info.num_subcores
    n_iters = (_SC_NCHUNKS + nw - 1) // nw
    wid = lax.axis_index("s") * info.num_cores + lax.axis_index("c")

    def chunk_body(it, carry):
        chunk = wid + it * nw

        @pl.when(chunk < _SC_NCHUNKS)
        def _():
            base_r = chunk * _SC_CH
            base_s = base_r * K_SAMP
            pltpu.sync_copy(idx_hbm.at[pl.ds(base_s, _SC_S)], idx_v)
            pltpu.sync_copy(wt_hbm.at[pl.ds(base_s, _SC_S)], wt_v)
            pltpu.async_copy(table_hbm.at[idx_v], rows_v, sem).wait()

            def row_body(r, c2):
                s0 = r * K_SAMP
                wv0 = wt_v[pl.ds(s0, 16)]
                wv1 = wt_v[pl.ds(s0 + 16, 16)]
                a0 = jnp.zeros((16,), jnp.float32)
                a1 = jnp.zeros((16,), jnp.float32)
                for k in range(K_SAMP):
                    w = wv0[k] if k < 16 else wv1[k - 16]
                    a0 = a0 + w * rows_v[s0 + k, pl.ds(0, 16)]
                    a1 = a1 + w * rows_v[s0 + k, pl.ds(16, 16)]
                out_v[r, pl.ds(0, 16)] = a0
                out_v[r, pl.ds(16, 16)] = a1
                return c2
            lax.fori_loop(0, _SC_CH, row_body, 0)
            pltpu.sync_copy(out_v, out_hbm.at[pl.ds(base_r, _SC_CH)])
        return carry

    lax.fori_loop(0, n_iters, chunk_body, 0)


def _sc_gather(table, idx, wt):
    mesh = plsc.VectorSubcoreMesh(core_axis_name="c", subcore_axis_name="s")
    k = functools.partial(
        pl.kernel,
        mesh=mesh,
        compiler_params=pltpu.CompilerParams(use_tc_tiling_on_sc=False),
        out_type=jax.ShapeDtypeStruct((R_ROWS, HEAD_DIM), jnp.float32),
        scratch_types=[
            pltpu.VMEM((_SC_S,), jnp.int32),
            pltpu.VMEM((_SC_S,), jnp.float32),
            pltpu.VMEM((_SC_S, HEAD_DIM), jnp.float32),
            pltpu.VMEM((_SC_CH, HEAD_DIM), jnp.float32),
            pltpu.SemaphoreType.DMA,
        ],
    )(_sc_gather_kernel)
    return k(table, idx, wt)


def _c_body(o_ref, m_ref, q_ref, wo_ref, bo_ref, out_ref):
    msum = jnp.sum(m_ref[...], axis=2)
    vf = (msum > 0.0).astype(jnp.float32)
    acc = jnp.sum(o_ref[...] * vf[:, :, None], axis=0)
    cnt = jnp.maximum(jnp.sum(vf, axis=0), 1.0)
    qo = acc / cnt[:, None]
    out_ref[...] = (jnp.dot(qo, wo_ref[...], preferred_element_type=jnp.float32)
                    + bo_ref[...] + q_ref[...])


def _finalize(out_sc, maskf, query2d, W_out, b_out):
    q_blk = 2000
    grid = (NQ // q_blk,)
    res = pl.pallas_call(
        _c_body,
        grid=grid,
        in_specs=[
            pl.BlockSpec((N_CAM, q_blk, C), lambda i: (0, i, 0)),
            pl.BlockSpec((N_CAM, q_blk, Y), lambda i: (0, i, 0)),
            pl.BlockSpec((q_blk, C), lambda i: (i, 0)),
            pl.BlockSpec((C, C), lambda i: (0, 0)),
            pl.BlockSpec((1, C), lambda i: (0, 0)),
        ],
        out_specs=pl.BlockSpec((q_blk, C), lambda i: (i, 0)),
        out_shape=jax.ShapeDtypeStruct((NQ, C), jnp.float32),
    )(out_sc, maskf, query2d, W_out.T, b_out.reshape(1, C))
    return res


def kernel(query, key, value, query_pos, ref_pts_cam, spatial_shapes, bev_mask,
           W_value, b_value, W_off, b_off, W_attn, b_attn, W_out, b_out):
    del key, spatial_shapes
    query2d = query.reshape(NQ, C)
    qp2d = query_pos.reshape(NQ, C)
    val = jnp.transpose(value, (0, 2, 1, 3)).reshape(N_CAM, HW, C)

    table = _value_table(val, W_value, b_value)
    off, aw = _proj_off_aw(query2d, qp2d, W_off, b_off, W_attn, b_attn)
    idx, wt = _sample_idx_wt(off, aw, ref_pts_cam)
    out_sc = _sc_gather(table, idx, wt)

    out3 = out_sc.reshape(N_CAM, NQ, C)
    maskf = bev_mask.reshape(N_CAM, NQ, Y).astype(jnp.float32)
    res = _finalize(out3, maskf, query2d, W_out, b_out)
    return res.reshape(1, NQ, C)


# speedup vs baseline: 22.5493x; 1.0573x over previous
import functools

import jax
import jax.numpy as jnp
from jax import lax
from jax.experimental import pallas as pl
from jax.experimental.pallas import tpu as pltpu
from jax.experimental.pallas import tpu_sc as plsc

N_CAM = 6
NQ = 10000
C = 128
H = 50
W = 100
HW = H * W
Y = 4
NUM_HEADS = 4
NUM_POINTS = 8
HEAD_DIM = C // NUM_HEADS

R_ROWS = N_CAM * NQ * NUM_HEADS
K_SAMP = NUM_POINTS * 4
TBL_ROWS = N_CAM * NUM_HEADS * HW


def _a1_body(v_ref, wv_ref, bv_ref, out_ref):
    x = jnp.dot(v_ref[0], wv_ref[...], preferred_element_type=jnp.float32)
    x = x + bv_ref[...]
    for h in range(NUM_HEADS):
        out_ref[0, h] = x[:, h * HEAD_DIM:(h + 1) * HEAD_DIM]


def _value_table(value, W_value, b_value):
    hw_blk = 1000
    grid = (N_CAM, HW // hw_blk)
    out = pl.pallas_call(
        _a1_body,
        grid=grid,
        in_specs=[
            pl.BlockSpec((1, hw_blk, C), lambda c, i: (c, i, 0)),
            pl.BlockSpec((C, C), lambda c, i: (0, 0)),
            pl.BlockSpec((1, C), lambda c, i: (0, 0)),
        ],
        out_specs=pl.BlockSpec((1, NUM_HEADS, hw_blk, HEAD_DIM),
                               lambda c, i: (c, 0, i, 0)),
        out_shape=jax.ShapeDtypeStruct((N_CAM, NUM_HEADS, HW, HEAD_DIM),
                                       jnp.float32),
    )(value, W_value.T, b_value.reshape(1, C))
    return out.reshape(TBL_ROWS, HEAD_DIM)


def _a2_body(q_ref, qp_ref, wo_ref, bo_ref, wa_ref, ba_ref, off_ref, aw_ref):
    q2 = q_ref[...] + qp_ref[...]
    off_ref[...] = jnp.dot(q2, wo_ref[...],
                           preferred_element_type=jnp.float32) + bo_ref[...]
    a = jnp.dot(q2, wa_ref[...], preferred_element_type=jnp.float32) + ba_ref[...]
    for h in range(NUM_HEADS):
        s = a[:, h * NUM_POINTS:(h + 1) * NUM_POINTS]
        m = jnp.max(s, axis=1, keepdims=True)
        e = jnp.exp(s - m)
        aw_ref[:, h * NUM_POINTS:(h + 1) * NUM_POINTS] = (
            e / jnp.sum(e, axis=1, keepdims=True))


def _proj_off_aw(query2d, query_pos2d, W_off, b_off, W_attn, b_attn):
    q_blk = 2000
    grid = (NQ // q_blk,)
    noff = NUM_HEADS * NUM_POINTS * 2
    naw = NUM_HEADS * NUM_POINTS
    off, aw = pl.pallas_call(
        _a2_body,
        grid=grid,
        in_specs=[
            pl.BlockSpec((q_blk, C), lambda i: (i, 0)),
            pl.BlockSpec((q_blk, C), lambda i: (i, 0)),
            pl.BlockSpec((C, noff), lambda i: (0, 0)),
            pl.BlockSpec((1, noff), lambda i: (0, 0)),
            pl.BlockSpec((C, naw), lambda i: (0, 0)),
            pl.BlockSpec((1, naw), lambda i: (0, 0)),
        ],
        out_specs=[
            pl.BlockSpec((q_blk, noff), lambda i: (i, 0)),
            pl.BlockSpec((q_blk, naw), lambda i: (i, 0)),
        ],
        out_shape=[
            jax.ShapeDtypeStruct((NQ, noff), jnp.float32),
            jax.ShapeDtypeStruct((NQ, naw), jnp.float32),
        ],
    )(query2d, query_pos2d, W_off.T, b_off.reshape(1, noff),
      W_attn.T, b_attn.reshape(1, naw))
    return off, aw


def _sample_idx_wt(off, aw, ref_pts_cam):
    off_r = off.reshape(NQ, NUM_HEADS, NUM_POINTS, 2)
    off_r = off_r / jnp.array([float(W), float(H)], jnp.float32)
    rr = ref_pts_cam.reshape(N_CAM, NQ, Y, 2)
    rr_p = rr[:, :, jnp.arange(NUM_POINTS) % Y, :]
    loc = rr_p[:, :, None, :, :] + off_r[None]
    x = loc[..., 0] * W - 0.5
    y = loc[..., 1] * H - 0.5
    x0 = jnp.floor(x)
    y0 = jnp.floor(y)
    wx1 = x - x0
    wx0 = 1.0 - wx1
    wy1 = y - y0
    wy0 = 1.0 - wy1
    xi = jnp.stack([x0, x0 + 1.0, x0, x0 + 1.0], axis=-1)
    yi = jnp.stack([y0, y0, y0 + 1.0, y0 + 1.0], axis=-1)
    wc = jnp.stack([wx0 * wy0, wx1 * wy0, wx0 * wy1, wx1 * wy1], axis=-1)
    valid = ((xi >= 0.0) & (xi <= float(W - 1))
             & (yi >= 0.0) & (yi <= float(H - 1))).astype(jnp.float32)
    xc = jnp.clip(xi, 0.0, float(W - 1)).astype(jnp.int32)
    yc = jnp.clip(yi, 0.0, float(H - 1)).astype(jnp.int32)
    lidx = yc * W + xc
    cam_h = (jnp.arange(N_CAM, dtype=jnp.int32)[:, None, None, None, None]
             * NUM_HEADS
             + jnp.arange(NUM_HEADS, dtype=jnp.int32)[None, None, :, None, None])
    gidx = cam_h * HW + lidx
    wt = aw.reshape(NQ, NUM_HEADS, NUM_POINTS)[None, :, :, :, None] * wc * valid
    return gidx.reshape(-1), wt.reshape(-1)


_SC_CH = 48
_SC_S = _SC_CH * K_SAMP
_SC_NCHUNKS = R_ROWS // _SC_CH


def _sc_gather_kernel(table_hbm, idx_hbm, wt_hbm, out_hbm,
                      idx_v0, wt_v0, rows_v0, idx_v1, wt_v1, rows_v1,
                      out_v, sem0, sem1):
    info = plsc.get_sparse_core_info()
    nw = info.num_cores * info.num_subcores
    n_iters = (_SC_NCHUNKS + nw - 1) // nw
    n_pairs = (n_iters + 1) // 2
    wid = lax.axis_index("s") * info.num_cores + lax.axis_index("c")

    bufs = ((idx_v0, wt_v0, rows_v0, sem0), (idx_v1, wt_v1, rows_v1, sem1))

    def start(it, buf):
        idx_v, wt_v, rows_v, sem = buf
        chunk = wid + it * nw

        @pl.when(chunk < _SC_NCHUNKS)
        def _():
            base_s = chunk * _SC_S
            pltpu.sync_copy(idx_hbm.at[pl.ds(base_s, _SC_S)], idx_v)
            pltpu.sync_copy(wt_hbm.at[pl.ds(base_s, _SC_S)], wt_v)
            pltpu.async_copy(table_hbm.at[idx_v], rows_v, sem)

    def finish(it, buf):
        idx_v, wt_v, rows_v, sem = buf
        chunk = wid + it * nw

        @pl.when(chunk < _SC_NCHUNKS)
        def _():
            pltpu.make_async_copy(table_hbm.at[idx_v], rows_v, sem).wait()

            def row_body(r, c2):
                s0 = r * K_SAMP
                wv0 = wt_v[pl.ds(s0, 16)]
                wv1 = wt_v[pl.ds(s0 + 16, 16)]
                a0 = jnp.zeros((16,), jnp.float32)
                a1 = jnp.zeros((16,), jnp.float32)
                for k in range(K_SAMP):
                    w = wv0[k] if k < 16 else wv1[k - 16]
                    a0 = a0 + w * rows_v[s0 + k, pl.ds(0, 16)]
                    a1 = a1 + w * rows_v[s0 + k, pl.ds(16, 16)]
                out_v[r, pl.ds(0, 16)] = a0
                out_v[r, pl.ds(16, 16)] = a1
                return c2
            lax.fori_loop(0, _SC_CH, row_body, 0)
            pltpu.sync_copy(out_v, out_hbm.at[pl.ds(chunk * _SC_CH, _SC_CH)])

    start(0, bufs[0])

    def pair_body(p, carry):
        it0 = p * 2
        start(it0 + 1, bufs[1])
        finish(it0, bufs[0])
        start(it0 + 2, bufs[0])
        finish(it0 + 1, bufs[1])
        return carry

    lax.fori_loop(0, n_pairs, pair_body, 0)


def _sc_gather(table, idx, wt):
    mesh = plsc.VectorSubcoreMesh(core_axis_name="c", subcore_axis_name="s")
    k = functools.partial(
        pl.kernel,
        mesh=mesh,
        compiler_params=pltpu.CompilerParams(use_tc_tiling_on_sc=False),
        out_type=jax.ShapeDtypeStruct((R_ROWS, HEAD_DIM), jnp.float32),
        scratch_types=[
            pltpu.VMEM((_SC_S,), jnp.int32),
            pltpu.VMEM((_SC_S,), jnp.float32),
            pltpu.VMEM((_SC_S, HEAD_DIM), jnp.float32),
            pltpu.VMEM((_SC_S,), jnp.int32),
            pltpu.VMEM((_SC_S,), jnp.float32),
            pltpu.VMEM((_SC_S, HEAD_DIM), jnp.float32),
            pltpu.VMEM((_SC_CH, HEAD_DIM), jnp.float32),
            pltpu.SemaphoreType.DMA,
            pltpu.SemaphoreType.DMA,
        ],
    )(_sc_gather_kernel)
    return k(table, idx, wt)


def _c_body(o_ref, m_ref, q_ref, wo_ref, bo_ref, out_ref):
    msum = jnp.sum(m_ref[...], axis=2)
    vf = (msum > 0.0).astype(jnp.float32)
    acc = jnp.sum(o_ref[...] * vf[:, :, None], axis=0)
    cnt = jnp.maximum(jnp.sum(vf, axis=0), 1.0)
    qo = acc / cnt[:, None]
    out_ref[...] = (jnp.dot(qo, wo_ref[...], preferred_element_type=jnp.float32)
                    + bo_ref[...] + q_ref[...])


def _finalize(out_sc, maskf, query2d, W_out, b_out):
    q_blk = 2000
    grid = (NQ // q_blk,)
    res = pl.pallas_call(
        _c_body,
        grid=grid,
        in_specs=[
            pl.BlockSpec((N_CAM, q_blk, C), lambda i: (0, i, 0)),
            pl.BlockSpec((N_CAM, q_blk, Y), lambda i: (0, i, 0)),
            pl.BlockSpec((q_blk, C), lambda i: (i, 0)),
            pl.BlockSpec((C, C), lambda i: (0, 0)),
            pl.BlockSpec((1, C), lambda i: (0, 0)),
        ],
        out_specs=pl.BlockSpec((q_blk, C), lambda i: (i, 0)),
        out_shape=jax.ShapeDtypeStruct((NQ, C), jnp.float32),
    )(out_sc, maskf, query2d, W_out.T, b_out.reshape(1, C))
    return res


def kernel(query, key, value, query_pos, ref_pts_cam, spatial_shapes, bev_mask,
           W_value, b_value, W_off, b_off, W_attn, b_attn, W_out, b_out):
    del key, spatial_shapes
    query2d = query.reshape(NQ, C)
    qp2d = query_pos.reshape(NQ, C)
    val = jnp.transpose(value, (0, 2, 1, 3)).reshape(N_CAM, HW, C)

    table = _value_table(val, W_value, b_value)
    off, aw = _proj_off_aw(query2d, qp2d, W_off, b_off, W_attn, b_attn)
    idx, wt = _sample_idx_wt(off, aw, ref_pts_cam)
    out_sc = _sc_gather(table, idx, wt)

    out3 = out_sc.reshape(N_CAM, NQ, C)
    maskf = bev_mask.reshape(N_CAM, NQ, Y).astype(jnp.float32)
    res = _finalize(out3, maskf, query2d, W_out, b_out)
    return res.reshape(1, NQ, C)


# speedup vs baseline: 22.7241x; 1.0078x over previous
import functools

import jax
import jax.numpy as jnp
from jax import lax
from jax.experimental import pallas as pl
from jax.experimental.pallas import tpu as pltpu
from jax.experimental.pallas import tpu_sc as plsc

N_CAM = 6
NQ = 10000
C = 128
H = 50
W = 100
HW = H * W
Y = 4
NUM_HEADS = 4
NUM_POINTS = 8
HEAD_DIM = C // NUM_HEADS

R_ROWS = N_CAM * NQ * NUM_HEADS
K_SAMP = NUM_POINTS * 4
TBL_ROWS = N_CAM * NUM_HEADS * HW


def _a1_body(v_ref, wv_ref, bv_ref, out_ref):
    x = jnp.dot(v_ref[0], wv_ref[...], preferred_element_type=jnp.float32)
    x = x + bv_ref[...]
    for h in range(NUM_HEADS):
        out_ref[0, h] = x[:, h * HEAD_DIM:(h + 1) * HEAD_DIM]


def _value_table(value, W_value, b_value):
    hw_blk = 1000
    grid = (N_CAM, HW // hw_blk)
    out = pl.pallas_call(
        _a1_body,
        grid=grid,
        in_specs=[
            pl.BlockSpec((1, hw_blk, C), lambda c, i: (c, i, 0)),
            pl.BlockSpec((C, C), lambda c, i: (0, 0)),
            pl.BlockSpec((1, C), lambda c, i: (0, 0)),
        ],
        out_specs=pl.BlockSpec((1, NUM_HEADS, hw_blk, HEAD_DIM),
                               lambda c, i: (c, 0, i, 0)),
        out_shape=jax.ShapeDtypeStruct((N_CAM, NUM_HEADS, HW, HEAD_DIM),
                                       jnp.float32),
    )(value, W_value.T, b_value.reshape(1, C))
    return out.reshape(TBL_ROWS, HEAD_DIM)


def _a2_body(q_ref, qp_ref, wo_ref, bo_ref, wa_ref, ba_ref, off_ref, aw_ref):
    q2 = q_ref[...] + qp_ref[...]
    off_ref[...] = jnp.dot(q2, wo_ref[...],
                           preferred_element_type=jnp.float32) + bo_ref[...]
    a = jnp.dot(q2, wa_ref[...], preferred_element_type=jnp.float32) + ba_ref[...]
    for h in range(NUM_HEADS):
        s = a[:, h * NUM_POINTS:(h + 1) * NUM_POINTS]
        m = jnp.max(s, axis=1, keepdims=True)
        e = jnp.exp(s - m)
        aw_ref[:, h * NUM_POINTS:(h + 1) * NUM_POINTS] = (
            e / jnp.sum(e, axis=1, keepdims=True))


def _proj_off_aw(query2d, query_pos2d, W_off, b_off, W_attn, b_attn):
    q_blk = 2000
    grid = (NQ // q_blk,)
    noff = NUM_HEADS * NUM_POINTS * 2
    naw = NUM_HEADS * NUM_POINTS
    off, aw = pl.pallas_call(
        _a2_body,
        grid=grid,
        in_specs=[
            pl.BlockSpec((q_blk, C), lambda i: (i, 0)),
            pl.BlockSpec((q_blk, C), lambda i: (i, 0)),
            pl.BlockSpec((C, noff), lambda i: (0, 0)),
            pl.BlockSpec((1, noff), lambda i: (0, 0)),
            pl.BlockSpec((C, naw), lambda i: (0, 0)),
            pl.BlockSpec((1, naw), lambda i: (0, 0)),
        ],
        out_specs=[
            pl.BlockSpec((q_blk, noff), lambda i: (i, 0)),
            pl.BlockSpec((q_blk, naw), lambda i: (i, 0)),
        ],
        out_shape=[
            jax.ShapeDtypeStruct((NQ, noff), jnp.float32),
            jax.ShapeDtypeStruct((NQ, naw), jnp.float32),
        ],
    )(query2d, query_pos2d, W_off.T, b_off.reshape(1, noff),
      W_attn.T, b_attn.reshape(1, naw))
    return off, aw


def _sample_idx_wt(off, aw, ref_pts_cam):
    off_r = off.reshape(NQ, NUM_HEADS, NUM_POINTS, 2)
    off_x = off_r[..., 0] / float(W)
    off_y = off_r[..., 1] / float(H)
    rr = ref_pts_cam.reshape(N_CAM, NQ, 1, Y, 2)
    rr_b = jnp.broadcast_to(rr[:, :, :, None, :, :],
                            (N_CAM, NQ, 1, NUM_POINTS // Y, Y, 2))
    rr_b = rr_b.reshape(N_CAM, NQ, 1, NUM_POINTS, 2)
    x = (rr_b[..., 0] + off_x[None]) * W - 0.5
    y = (rr_b[..., 1] + off_y[None]) * H - 0.5
    x0 = jnp.floor(x)[..., None]
    y0 = jnp.floor(y)[..., None]
    wx1 = x[..., None] - x0
    wy1 = y[..., None] - y0
    dxc = jnp.array([0.0, 1.0, 0.0, 1.0], jnp.float32)
    dyc = jnp.array([0.0, 0.0, 1.0, 1.0], jnp.float32)
    xi = x0 + dxc
    yi = y0 + dyc
    wc = (dxc * wx1 + (1.0 - dxc) * (1.0 - wx1)) * (
        dyc * wy1 + (1.0 - dyc) * (1.0 - wy1))
    valid = ((xi >= 0.0) & (xi <= float(W - 1))
             & (yi >= 0.0) & (yi <= float(H - 1))).astype(jnp.float32)
    xc = jnp.clip(xi, 0.0, float(W - 1)).astype(jnp.int32)
    yc = jnp.clip(yi, 0.0, float(H - 1)).astype(jnp.int32)
    lidx = yc * W + xc
    cam_h = (jnp.arange(N_CAM, dtype=jnp.int32)[:, None, None, None, None]
             * NUM_HEADS
             + jnp.arange(NUM_HEADS, dtype=jnp.int32)[None, None, :, None, None])
    gidx = cam_h * HW + lidx
    wt = aw.reshape(NQ, NUM_HEADS, NUM_POINTS)[None, :, :, :, None] * wc * valid
    return gidx.reshape(-1), wt.reshape(-1)


_SC_CH = 48
_SC_S = _SC_CH * K_SAMP
_SC_NCHUNKS = R_ROWS // _SC_CH


def _sc_gather_kernel(table_hbm, idx_hbm, wt_hbm, out_hbm,
                      idx_v0, wt_v0, rows_v0, idx_v1, wt_v1, rows_v1,
                      out_v, sem0, sem1):
    info = plsc.get_sparse_core_info()
    nw = info.num_cores * info.num_subcores
    n_iters = (_SC_NCHUNKS + nw - 1) // nw
    n_pairs = (n_iters + 1) // 2
    wid = lax.axis_index("s") * info.num_cores + lax.axis_index("c")

    bufs = ((idx_v0, wt_v0, rows_v0, sem0), (idx_v1, wt_v1, rows_v1, sem1))

    def start(it, buf):
        idx_v, wt_v, rows_v, sem = buf
        chunk = wid + it * nw

        @pl.when(chunk < _SC_NCHUNKS)
        def _():
            base_s = chunk * _SC_S
            pltpu.sync_copy(idx_hbm.at[pl.ds(base_s, _SC_S)], idx_v)
            pltpu.sync_copy(wt_hbm.at[pl.ds(base_s, _SC_S)], wt_v)
            pltpu.async_copy(table_hbm.at[idx_v], rows_v, sem)

    def finish(it, buf):
        idx_v, wt_v, rows_v, sem = buf
        chunk = wid + it * nw

        @pl.when(chunk < _SC_NCHUNKS)
        def _():
            pltpu.make_async_copy(table_hbm.at[idx_v], rows_v, sem).wait()

            def row_body(r, c2):
                s0 = r * K_SAMP
                wv0 = wt_v[pl.ds(s0, 16)]
                wv1 = wt_v[pl.ds(s0 + 16, 16)]
                a0 = jnp.zeros((16,), jnp.float32)
                a1 = jnp.zeros((16,), jnp.float32)
                for k in range(K_SAMP):
                    w = wv0[k] if k < 16 else wv1[k - 16]
                    a0 = a0 + w * rows_v[s0 + k, pl.ds(0, 16)]
                    a1 = a1 + w * rows_v[s0 + k, pl.ds(16, 16)]
                out_v[r, pl.ds(0, 16)] = a0
                out_v[r, pl.ds(16, 16)] = a1
                return c2
            lax.fori_loop(0, _SC_CH, row_body, 0)
            pltpu.sync_copy(out_v, out_hbm.at[pl.ds(chunk * _SC_CH, _SC_CH)])

    start(0, bufs[0])

    def pair_body(p, carry):
        it0 = p * 2
        start(it0 + 1, bufs[1])
        finish(it0, bufs[0])
        start(it0 + 2, bufs[0])
        finish(it0 + 1, bufs[1])
        return carry

    lax.fori_loop(0, n_pairs, pair_body, 0)


def _sc_gather(table, idx, wt):
    mesh = plsc.VectorSubcoreMesh(core_axis_name="c", subcore_axis_name="s")
    k = functools.partial(
        pl.kernel,
        mesh=mesh,
        compiler_params=pltpu.CompilerParams(use_tc_tiling_on_sc=False),
        out_type=jax.ShapeDtypeStruct((R_ROWS, HEAD_DIM), jnp.float32),
        scratch_types=[
            pltpu.VMEM((_SC_S,), jnp.int32),
            pltpu.VMEM((_SC_S,), jnp.float32),
            pltpu.VMEM((_SC_S, HEAD_DIM), jnp.float32),
            pltpu.VMEM((_SC_S,), jnp.int32),
            pltpu.VMEM((_SC_S,), jnp.float32),
            pltpu.VMEM((_SC_S, HEAD_DIM), jnp.float32),
            pltpu.VMEM((_SC_CH, HEAD_DIM), jnp.float32),
            pltpu.SemaphoreType.DMA,
            pltpu.SemaphoreType.DMA,
        ],
    )(_sc_gather_kernel)
    return k(table, idx, wt)


def _c_body(o_ref, m_ref, q_ref, wo_ref, bo_ref, out_ref):
    msum = jnp.sum(m_ref[...], axis=2)
    vf = (msum > 0.0).astype(jnp.float32)
    acc = jnp.sum(o_ref[...] * vf[:, :, None], axis=0)
    cnt = jnp.maximum(jnp.sum(vf, axis=0), 1.0)
    qo = acc / cnt[:, None]
    out_ref[...] = (jnp.dot(qo, wo_ref[...], preferred_element_type=jnp.float32)
                    + bo_ref[...] + q_ref[...])


def _finalize(out_sc, maskf, query2d, W_out, b_out):
    q_blk = 2000
    grid = (NQ // q_blk,)
    res = pl.pallas_call(
        _c_body,
        grid=grid,
        in_specs=[
            pl.BlockSpec((N_CAM, q_blk, C), lambda i: (0, i, 0)),
            pl.BlockSpec((N_CAM, q_blk, Y), lambda i: (0, i, 0)),
            pl.BlockSpec((q_blk, C), lambda i: (i, 0)),
            pl.BlockSpec((C, C), lambda i: (0, 0)),
            pl.BlockSpec((1, C), lambda i: (0, 0)),
        ],
        out_specs=pl.BlockSpec((q_blk, C), lambda i: (i, 0)),
        out_shape=jax.ShapeDtypeStruct((NQ, C), jnp.float32),
    )(out_sc, maskf, query2d, W_out.T, b_out.reshape(1, C))
    return res


def kernel(query, key, value, query_pos, ref_pts_cam, spatial_shapes, bev_mask,
           W_value, b_value, W_off, b_off, W_attn, b_attn, W_out, b_out):
    del key, spatial_shapes
    query2d = query.reshape(NQ, C)
    qp2d = query_pos.reshape(NQ, C)
    val = jnp.transpose(value, (0, 2, 1, 3)).reshape(N_CAM, HW, C)

    table = _value_table(val, W_value, b_value)
    off, aw = _proj_off_aw(query2d, qp2d, W_off, b_off, W_attn, b_attn)
    idx, wt = _sample_idx_wt(off, aw, ref_pts_cam)
    out_sc = _sc_gather(table, idx, wt)

    out3 = out_sc.reshape(N_CAM, NQ, C)
    maskf = bev_mask.reshape(N_CAM, NQ, Y).astype(jnp.float32)
    res = _finalize(out3, maskf, query2d, W_out, b_out)
    return res.reshape(1, NQ, C)
